# baseline (speedup 1.0000x reference)
import functools

import jax
import jax.numpy as jnp
from jax import lax
from jax.experimental import pallas as pl
from jax.experimental.pallas import tpu as pltpu
from jax.experimental.pallas import tpu_sc as plsc

V = 50000
D = 128
N_CTX = 50
B, LQ, N, LC = 4, 32, 20000, 32

N_PAD = 20480
PARTS = 8
SENT_PER_W = N_PAD // PARTS
GROUPS_PER_W = SENT_PER_W // 16
CHUNK_GROUPS = 32
N_CHUNKS = GROUPS_PER_W // CHUNK_GROUPS
VB = 6272
V_T = 50176
SENT_PER_PART = 8
TOK = SENT_PER_PART * LC

_MESH = plsc.VectorSubcoreMesh(core_axis_name="c", subcore_axis_name="s")
_SC_PARAMS = pltpu.CompilerParams(needs_layout_passes=False)
NEG = -3e38


def _wid():
    return lax.axis_index("s") * 2 + lax.axis_index("c")


def _qgather_sc(qw, qf, emb_ir, emb_rc, emb_feat):
    @functools.partial(
        pl.kernel,
        out_type=jax.ShapeDtypeStruct((3, B * LQ, D), jnp.float32),
        mesh=_MESH,
        scratch_types=[
            pltpu.VMEM((LQ,), jnp.int32),
            pltpu.VMEM((LQ, D), jnp.float32),
            pltpu.SemaphoreType.DMA,
        ],
    )
    def k(qw_hbm, qf_hbm, ir_hbm, rc_hbm, ft_hbm, out_hbm, idx_v, rows_v, sem):
        wid = _wid()
        for tt, (ids_hbm, tab) in enumerate(
            [(qw_hbm, ir_hbm), (qw_hbm, rc_hbm), (qf_hbm, ft_hbm)]
        ):
            @pl.when((wid >= tt * B) & (wid < tt * B + B))
            def _(tt=tt, ids_hbm=ids_hbm, tab=tab):
                b = wid - tt * B
                pltpu.sync_copy(ids_hbm.at[b], idx_v)
                pltpu.async_copy(tab.at[idx_v], rows_v, sem).wait()
                pltpu.sync_copy(rows_v, out_hbm.at[tt, pl.ds(b * LQ, LQ)])

    return k(qw, qf, emb_ir, emb_rc, emb_feat)


def _prep_tc(qe, qm, inv_q, W_s, emb_ir):
    def body(qe_ref, qm_ref, inv_ref, ws_ref, emb_ref, T_ref, u_ref, qp_scr):
        i = pl.program_id(0)

        @pl.when(i == 0)
        def _():
            qe_all = qe_ref[...]
            qm3 = qm_ref[...][:, :, None]
            inv = inv_ref[...]
            qe_ir = qe_all[0].reshape(B, LQ, D)
            qe_rc = (qe_all[1] + qe_all[2]).reshape(B, LQ, D)
            qvec = (qe_ir * qm3).sum(1) * inv
            qv2 = (qe_rc * qm3).sum(1) * inv
            zpad = jnp.zeros((8 - B, D), jnp.float32)
            qp_scr[...] = jnp.concatenate([qvec, zpad], 0)
            uu = lax.dot_general(qv2, ws_ref[...], (((1,), (1,)), ((), ())))
            u_ref[...] = jnp.concatenate([uu, zpad], 0)

        cols = lax.broadcasted_iota(jnp.int32, (8, VB), 1) + i * VB
        vals = lax.dot_general(
            qp_scr[...], emb_ref[...], (((1,), (1,)), ((), ()))
        )
        T_ref[...] = jnp.where(cols < V, vals, 0.0)

    return pl.pallas_call(
        body,
        grid=(V_T // VB,),
        in_specs=[
            pl.BlockSpec((3, B * LQ, D), lambda i: (0, 0, 0)),
            pl.BlockSpec((B, LQ), lambda i: (0, 0)),
            pl.BlockSpec((B, 1), lambda i: (0, 0)),
            pl.BlockSpec((D, D), lambda i: (0, 0)),
            pl.BlockSpec((VB, D), lambda i: (i, 0)),
        ],
        out_specs=[
            pl.BlockSpec((8, VB), lambda i: (0, i)),
            pl.BlockSpec((8, D), lambda i: (0, 0)),
        ],
        out_shape=[
            jax.ShapeDtypeStruct((8, V_T), jnp.float32),
            jax.ShapeDtypeStruct((8, D), jnp.float32),
        ],
        scratch_shapes=[pltpu.VMEM((8, D), jnp.float32)],
    )(qe, qm, inv_q, W_s, emb_ir)


def _score_sc(T, c0t, inv_pad):
    CG16 = CHUNK_GROUPS * 16
    CROWS = CG16 // 4

    @functools.partial(
        pl.kernel,
        out_type=jax.ShapeDtypeStruct((B, N_PAD), jnp.float32),
        mesh=_MESH,
        scratch_types=[
            pltpu.VMEM((V_T,), jnp.float32),
            pltpu.VMEM((2, CHUNK_GROUPS, LC * 16), jnp.int32),
            pltpu.VMEM((2, CG16), jnp.float32),
            pltpu.VMEM((CG16,), jnp.float32),
            pltpu.SemaphoreType.DMA,
            pltpu.SemaphoreType.DMA,
        ],
        compiler_params=_SC_PARAMS,
    )
    def k(T_hbm, c0t_hbm, inv_hbm, out_hbm, tb_v, c_v, inv_v, o_v,
          sem0, sem1):
        wid = _wid()
        b = wid // PARTS
        part = wid % PARTS
        pltpu.sync_copy(T_hbm.at[b], tb_v)
        base_g = part * GROUPS_PER_W
        sems = [sem0, sem1]

        def start(ci):
            g0 = base_g + ci * CHUNK_GROUPS
            buf = ci % 2
            return (
                pltpu.async_copy(
                    c0t_hbm.at[pl.ds(g0, CHUNK_GROUPS)], c_v.at[buf],
                    sems[buf]),
                pltpu.async_copy(
                    inv_hbm.at[pl.ds(g0 * 16, CG16)], inv_v.at[buf],
                    sems[buf]),
            )

        pend = start(0)
        for ci in range(N_CHUNKS):
            buf = ci % 2
            pend[0].wait()
            pend[1].wait()
            if ci + 1 < N_CHUNKS:
                pend = start(ci + 1)

            @pl.loop(0, CHUNK_GROUPS)
            def _grp(gl, buf=buf):
                acc0 = jnp.zeros((16,), jnp.float32)
                acc1 = jnp.zeros((16,), jnp.float32)
                for l in range(0, LC, 2):
                    acc0 = acc0 + plsc.load_gather(
                        tb_v, [c_v[buf, gl, pl.ds(l * 16, 16)]])
                    acc1 = acc1 + plsc.load_gather(
                        tb_v, [c_v[buf, gl, pl.ds((l + 1) * 16, 16)]])
                o_v[pl.ds(gl * 16, 16)] = (acc0 + acc1) * inv_v[
                    buf, pl.ds(gl * 16, 16)]

            col0 = part * SENT_PER_W + ci * CG16
            pltpu.sync_copy(o_v, out_hbm.at[b, pl.ds(col0, CG16)])

    return k(T, c0t, inv_pad)


def _rf_tc(u, emb_rc, emb_feat):
    def body(u_ref, rc_ref, ft_ref, R_ref, F_ref):
        uu = u_ref[...]
        R_ref[...] = lax.dot_general(uu, rc_ref[...], (((1,), (1,)), ((), ())))
        F_ref[...] = lax.dot_general(uu, ft_ref[...], (((1,), (1,)), ((), ())))

    return pl.pallas_call(
        body,
        grid=(V_T // VB,),
        in_specs=[
            pl.BlockSpec((8, D), lambda i: (0, 0)),
            pl.BlockSpec((VB, D), lambda i: (i, 0)),
            pl.BlockSpec((VB, D), lambda i: (i, 0)),
        ],
        out_specs=[
            pl.BlockSpec((8, VB), lambda i: (0, i)),
            pl.BlockSpec((8, VB), lambda i: (0, i)),
        ],
        out_shape=[
            jax.ShapeDtypeStruct((8, V_T), jnp.float32),
            jax.ShapeDtypeStruct((8, V_T), jnp.float32),
        ],
    )(u, emb_rc, emb_feat)


def _topk_tc(scores3):
    R = N_PAD // 128

    def body(s_ref, idx_ref, scr):
        lin3 = (
            lax.broadcasted_iota(jnp.int32, (B, R, 128), 1) * 128
            + lax.broadcasted_iota(jnp.int32, (B, R, 128), 2)
        )
        scr[...] = jnp.where(lin3 < N, s_ref[...], NEG)
        lane64 = lax.broadcasted_iota(jnp.int32, (B, 64), 1)

        def step(j, idxacc):
            cur = scr[...]
            m4 = jnp.max(cur, axis=(1, 2))
            cand = jnp.where(
                cur == m4[:, None, None], lin3, jnp.int32(2147483647))
            amin4 = jnp.min(cand, axis=(1, 2))
            scr[...] = jnp.where(lin3 == amin4[:, None, None], NEG, cur)
            return jnp.where(lane64 == j, amin4[:, None], idxacc)

        idx_ref[...] = lax.fori_loop(
            0, N_CTX, step, jnp.zeros((B, 64), jnp.int32))

    return pl.pallas_call(
        body,
        in_specs=[pl.BlockSpec((B, R, 128), lambda: (0, 0, 0))],
        out_shape=jax.ShapeDtypeStruct((B, 64), jnp.int32),
        scratch_shapes=[pltpu.VMEM((B, R, 128), jnp.float32)],
    )(scores3)


def _logits_sc(idx50, clen_pad, c2, a0, u, emb_rc, emb_feat):
    @functools.partial(
        pl.kernel,
        out_type=jax.ShapeDtypeStruct((B, PARTS, 16), jnp.float32),
        mesh=_MESH,
        scratch_types=[
            pltpu.VMEM((64,), jnp.int32),
            pltpu.VMEM((16,), jnp.int32),
            pltpu.VMEM((N_PAD,), jnp.int32),
            pltpu.VMEM((SENT_PER_PART, 128), jnp.int32),
            pltpu.VMEM((TOK,), jnp.int32),
            pltpu.VMEM((TOK,), jnp.int32),
            pltpu.VMEM((TOK, D), jnp.float32),
            pltpu.VMEM((TOK, D), jnp.float32),
            pltpu.VMEM((D,), jnp.float32),
            pltpu.VMEM((TOK,), jnp.float32),
            pltpu.VMEM((80,), jnp.int32),
            pltpu.VMEM((80,), jnp.int32),
            pltpu.VMEM((16,), jnp.int32),
            pltpu.VMEM((16,), jnp.float32),
            pltpu.SemaphoreType.DMA,
        ],
        compiler_params=_SC_PARAMS,
    )
    def k(idx_hbm, clen_hbm, c2_hbm, a_hbm, u_hbm, rc_hbm, ft_hbm,
          out_hbm, idx_v, idx2_v, clen_v, id_rows, wflat, fflat, erc, eft,
          ubuf, zbuf, clbuf, stbuf, a_v, outv, sem):
        wid = _wid()
        b = wid // PARTS
        part = wid % PARTS
        pltpu.sync_copy(idx_hbm.at[b], idx_v)
        pltpu.sync_copy(clen_hbm, clen_v)
        pltpu.sync_copy(u_hbm.at[b], ubuf)
        pltpu.sync_copy(a_hbm, a_v)

        i16 = lax.broadcasted_iota(jnp.int32, (16,), 0)
        myidx = plsc.load_gather(idx_v, [jnp.minimum(part * 8 + i16, 63)])
        idx2_v[...] = myidx // 2
        parv = myidx % 2
        sent2 = idx2_v.at[pl.ds(0, 8)]
        pltpu.async_copy(c2_hbm.at[sent2], id_rows, sem).wait()

        for r in range(SENT_PER_PART):
            pr = jnp.sum(jnp.where(i16 == r, parv, 0)) * 64
            for h in range(2):
                lanes = pr + 2 * (i16 + h * 16)
                rr = jnp.full((16,), r, jnp.int32)
                wflat[pl.ds(r * LC + h * 16, 16)] = plsc.load_gather(
                    id_rows, [rr, lanes])
                fflat[pl.ds(r * LC + h * 16, 16)] = plsc.load_gather(
                    id_rows, [rr, lanes + 1])
        pltpu.async_copy(rc_hbm.at[wflat], erc, sem).wait()
        pltpu.async_copy(ft_hbm.at[fflat], eft, sem).wait()

        carry = jnp.int32(0)
        for kk in range(4):
            idv = idx_v[pl.ds(kk * 16, 16)]
            cl = plsc.load_gather(clen_v, [idv])
            cl = jnp.where(i16 + kk * 16 < N_CTX, cl, 0)
            inc = plsc.cumsum(cl)
            clbuf[pl.ds(kk * 16, 16)] = cl
            stbuf[pl.ds(kk * 16, 16)] = inc - cl + carry
            carry = carry + jnp.sum(cl)

        uv = [ubuf[pl.ds(h * 16, 16)] for h in range(8)]

        @pl.loop(0, TOK // 16)
        def _tok(g):
            zvec = jnp.zeros((16,), jnp.float32)
            for tloc in range(16):
                t = g * 16 + tloc
                acc = jnp.zeros((16,), jnp.float32)
                for h in range(8):
                    sl = pl.ds(h * 16, 16)
                    acc = acc + (erc[t, sl] + eft[t, sl]) * uv[h]
                zvec = zvec + jnp.where(i16 == tloc, jnp.sum(acc), 0.0)
            zbuf[pl.ds(g * 16, 16)] = zvec

        mybase = part * SENT_PER_PART
        clv = clbuf[pl.ds(mybase, 16)]
        stv = stbuf[pl.ds(mybase, 16)]

        runmax = jnp.full((16,), NEG, jnp.float32)
        for jj in range(SENT_PER_PART):
            lenj = clv[jj]
            for h in range(2):
                zv = zbuf[pl.ds(jj * LC + h * 16, 16)]
                valid = i16 + h * 16 < lenj
                runmax = jnp.maximum(runmax, jnp.where(valid, zv, NEG))
        m_loc = jnp.max(runmax)
        sacc = jnp.zeros((16,), jnp.float32)
        for jj in range(SENT_PER_PART):
            lenj = clv[jj]
            for h in range(2):
                zv = zbuf[pl.ds(jj * LC + h * 16, 16)]
                valid = i16 + h * 16 < lenj
                sacc = sacc + jnp.where(valid, jnp.exp(zv - m_loc), 0.0)
        s_loc = jnp.sum(sacc)

        av = a_v[...]
        p = jnp.sum(jnp.where(i16 == b, av, 0))
        zpart = jnp.float32(0.0)
        for jj in range(SENT_PER_PART):
            stj = stv[jj]
            lenj = clv[jj]
            own = (p >= stj) & (p < stj + lenj)
            off = p - stj
            zv0 = zbuf[pl.ds(jj * LC, 16)]
            zv1 = zbuf[pl.ds(jj * LC + 16, 16)]
            zt = (jnp.sum(jnp.where(i16 == off, zv0, 0.0))
                  + jnp.sum(jnp.where(i16 == off - 16, zv1, 0.0)))
            zpart = zpart + jnp.where(own, zt, 0.0)

        ov = jnp.where(
            i16 == 0, m_loc,
            jnp.where(i16 == 1, s_loc, jnp.where(i16 == 2, zpart, 0.0)),
        )
        outv[...] = ov
        pltpu.sync_copy(outv, out_hbm.at[b, part])

    return k(idx50, clen_pad, c2, a0, u, emb_rc, emb_feat)


def _final_tc(e):
    def body(e_ref, o_ref):
        ev = e_ref[...]
        m = ev[:, :, 0]
        s = ev[:, :, 1]
        z = ev[:, :, 2]
        M = jnp.max(m, axis=1, keepdims=True)
        sm = (s * jnp.exp(m - M)).sum(1, keepdims=True)
        lse = jnp.log(sm) + M
        zt = z.sum(1, keepdims=True)
        o_ref[0, 0] = jnp.mean(lse - zt)

    return pl.pallas_call(
        body,
        in_specs=[pl.BlockSpec((B, PARTS, 16), lambda: (0, 0, 0))],
        out_specs=pl.BlockSpec(memory_space=pltpu.SMEM),
        out_shape=jax.ShapeDtypeStruct((1, 1), jnp.float32),
    )(e)


def kernel(q, c, a, qlen, clen, alen, c_batch_size, emb_ir, emb_rc, emb_feat,
           W_s, W_e):
    q = q.astype(jnp.int32)
    c = c.astype(jnp.int32)
    qlen = jnp.maximum(qlen.astype(jnp.int32), 1)
    clen = jnp.maximum(clen.astype(jnp.int32), 1)

    qw = q[:, :, 0]
    qf = q[:, :, 1]
    qe = _qgather_sc(qw, qf, emb_ir, emb_rc, emb_feat)

    qm = (jnp.arange(LQ)[None, :] < qlen[:, None]).astype(jnp.float32)
    inv_q = 1.0 / qm.sum(1, keepdims=True)
    T, u = _prep_tc(qe, qm, inv_q, W_s, emb_ir)

    c0 = c[:, :, 0]
    c0p = jnp.pad(c0, ((0, N_PAD - N), (0, 0)))
    clen_pad = jnp.pad(clen, (0, N_PAD - N), constant_values=1)
    c0m = jnp.where(jnp.arange(LC)[None, :] < clen_pad[:, None], c0p, V)
    c0t = c0m.reshape(N_PAD // 16, 16, LC).transpose(0, 2, 1).reshape(
        N_PAD // 16, LC * 16)

    inv_pad = 1.0 / clen_pad.astype(jnp.float32)
    scores = _score_sc(T, c0t, inv_pad)
    idx50 = _topk_tc(scores.reshape(B, N_PAD // 128, 128))

    a0 = jnp.pad(a[:, 0].astype(jnp.int32), (0, 16 - B))
    c2 = c.reshape(N // 2, 2 * LC * 2)
    e = _logits_sc(idx50, clen_pad, c2, a0, u, emb_rc, emb_feat)
    return _final_tc(e).reshape(())


# speedup vs baseline: 3.3027x; 3.3027x over previous
import functools

import jax
import jax.numpy as jnp
from jax import lax
from jax.experimental import pallas as pl
from jax.experimental.pallas import tpu as pltpu
from jax.experimental.pallas import tpu_sc as plsc

V = 50000
D = 128
N_CTX = 50
B, LQ, N, LC = 4, 32, 20000, 32

N_PAD = 20480
PARTS = 8
SENT_PER_W = N_PAD // PARTS
GROUPS_PER_W = SENT_PER_W // 16
CHUNK_GROUPS = 32
N_CHUNKS = GROUPS_PER_W // CHUNK_GROUPS
VB = 6272
V_T = 50176
SENT_PER_PART = 8
TOK = SENT_PER_PART * LC

_MESH = plsc.VectorSubcoreMesh(core_axis_name="c", subcore_axis_name="s")
_SC_PARAMS = pltpu.CompilerParams(needs_layout_passes=False)
NEG = -3e38


def _wid():
    return lax.axis_index("s") * 2 + lax.axis_index("c")


def _qgather_sc(qw, qf, emb_ir, emb_rc, emb_feat):
    @functools.partial(
        pl.kernel,
        out_type=jax.ShapeDtypeStruct((3, B * LQ, D), jnp.float32),
        mesh=_MESH,
        scratch_types=[
            pltpu.VMEM((LQ,), jnp.int32),
            pltpu.VMEM((LQ, D), jnp.float32),
            pltpu.SemaphoreType.DMA,
        ],
    )
    def k(qw_hbm, qf_hbm, ir_hbm, rc_hbm, ft_hbm, out_hbm, idx_v, rows_v, sem):
        wid = _wid()
        for tt, (ids_hbm, tab) in enumerate(
            [(qw_hbm, ir_hbm), (qw_hbm, rc_hbm), (qf_hbm, ft_hbm)]
        ):
            @pl.when((wid >= tt * B) & (wid < tt * B + B))
            def _(tt=tt, ids_hbm=ids_hbm, tab=tab):
                b = wid - tt * B
                pltpu.sync_copy(ids_hbm.at[b], idx_v)
                pltpu.async_copy(tab.at[idx_v], rows_v, sem).wait()
                pltpu.sync_copy(rows_v, out_hbm.at[tt, pl.ds(b * LQ, LQ)])

    return k(qw, qf, emb_ir, emb_rc, emb_feat)


def _prep_tc(qe, qm, inv_q, W_s, emb_ir):
    def body(qe_ref, qm_ref, inv_ref, ws_ref, emb_ref, T_ref, u_ref, qp_scr):
        i = pl.program_id(0)

        @pl.when(i == 0)
        def _():
            qe_all = qe_ref[...]
            qm3 = qm_ref[...][:, :, None]
            inv = inv_ref[...]
            qe_ir = qe_all[0].reshape(B, LQ, D)
            qe_rc = (qe_all[1] + qe_all[2]).reshape(B, LQ, D)
            qvec = (qe_ir * qm3).sum(1) * inv
            qv2 = (qe_rc * qm3).sum(1) * inv
            zpad = jnp.zeros((8 - B, D), jnp.float32)
            qp_scr[...] = jnp.concatenate([qvec, zpad], 0)
            uu = lax.dot_general(qv2, ws_ref[...], (((1,), (1,)), ((), ())))
            u_ref[...] = jnp.concatenate([uu, zpad], 0)

        cols = lax.broadcasted_iota(jnp.int32, (8, VB), 1) + i * VB
        vals = lax.dot_general(
            qp_scr[...], emb_ref[...], (((1,), (1,)), ((), ()))
        )
        T_ref[...] = jnp.where(cols < V, vals, 0.0)

    return pl.pallas_call(
        body,
        grid=(V_T // VB,),
        in_specs=[
            pl.BlockSpec((3, B * LQ, D), lambda i: (0, 0, 0)),
            pl.BlockSpec((B, LQ), lambda i: (0, 0)),
            pl.BlockSpec((B, 1), lambda i: (0, 0)),
            pl.BlockSpec((D, D), lambda i: (0, 0)),
            pl.BlockSpec((VB, D), lambda i: (i, 0)),
        ],
        out_specs=[
            pl.BlockSpec((8, VB), lambda i: (0, i)),
            pl.BlockSpec((8, D), lambda i: (0, 0)),
        ],
        out_shape=[
            jax.ShapeDtypeStruct((8, V_T), jnp.float32),
            jax.ShapeDtypeStruct((8, D), jnp.float32),
        ],
        scratch_shapes=[pltpu.VMEM((8, D), jnp.float32)],
    )(qe, qm, inv_q, W_s, emb_ir)


def _score_sc(T, c0t, inv_pad):
    CG16 = CHUNK_GROUPS * 16
    CROWS = CG16 // 4

    @functools.partial(
        pl.kernel,
        out_type=jax.ShapeDtypeStruct((B, N_PAD), jnp.float32),
        mesh=_MESH,
        scratch_types=[
            pltpu.VMEM((V_T,), jnp.float32),
            pltpu.VMEM((2, CHUNK_GROUPS, LC * 16), jnp.int32),
            pltpu.VMEM((2, CG16), jnp.float32),
            pltpu.VMEM((CG16,), jnp.float32),
            pltpu.SemaphoreType.DMA,
            pltpu.SemaphoreType.DMA,
        ],
        compiler_params=_SC_PARAMS,
    )
    def k(T_hbm, c0t_hbm, inv_hbm, out_hbm, tb_v, c_v, inv_v, o_v,
          sem0, sem1):
        wid = _wid()
        b = wid // PARTS
        part = wid % PARTS
        pltpu.sync_copy(T_hbm.at[b], tb_v)
        base_g = part * GROUPS_PER_W
        sems = [sem0, sem1]

        def start(ci):
            g0 = base_g + ci * CHUNK_GROUPS
            buf = ci % 2
            return (
                pltpu.async_copy(
                    c0t_hbm.at[pl.ds(g0, CHUNK_GROUPS)], c_v.at[buf],
                    sems[buf]),
                pltpu.async_copy(
                    inv_hbm.at[pl.ds(g0 * 16, CG16)], inv_v.at[buf],
                    sems[buf]),
            )

        pend = start(0)
        for ci in range(N_CHUNKS):
            buf = ci % 2
            pend[0].wait()
            pend[1].wait()
            if ci + 1 < N_CHUNKS:
                pend = start(ci + 1)

            @pl.loop(0, CHUNK_GROUPS)
            def _grp(gl, buf=buf):
                acc0 = jnp.zeros((16,), jnp.float32)
                acc1 = jnp.zeros((16,), jnp.float32)
                for l in range(0, LC, 2):
                    acc0 = acc0 + plsc.load_gather(
                        tb_v, [c_v[buf, gl, pl.ds(l * 16, 16)]])
                    acc1 = acc1 + plsc.load_gather(
                        tb_v, [c_v[buf, gl, pl.ds((l + 1) * 16, 16)]])
                o_v[pl.ds(gl * 16, 16)] = (acc0 + acc1) * inv_v[
                    buf, pl.ds(gl * 16, 16)]

            col0 = part * SENT_PER_W + ci * CG16
            pltpu.sync_copy(o_v, out_hbm.at[b, pl.ds(col0, CG16)])

    return k(T, c0t, inv_pad)


def _rf_tc(u, emb_rc, emb_feat):
    def body(u_ref, rc_ref, ft_ref, R_ref, F_ref):
        uu = u_ref[...]
        R_ref[...] = lax.dot_general(uu, rc_ref[...], (((1,), (1,)), ((), ())))
        F_ref[...] = lax.dot_general(uu, ft_ref[...], (((1,), (1,)), ((), ())))

    return pl.pallas_call(
        body,
        grid=(V_T // VB,),
        in_specs=[
            pl.BlockSpec((8, D), lambda i: (0, 0)),
            pl.BlockSpec((VB, D), lambda i: (i, 0)),
            pl.BlockSpec((VB, D), lambda i: (i, 0)),
        ],
        out_specs=[
            pl.BlockSpec((8, VB), lambda i: (0, i)),
            pl.BlockSpec((8, VB), lambda i: (0, i)),
        ],
        out_shape=[
            jax.ShapeDtypeStruct((8, V_T), jnp.float32),
            jax.ShapeDtypeStruct((8, V_T), jnp.float32),
        ],
    )(u, emb_rc, emb_feat)


def _topk_tc(scores3):
    R = N_PAD // 128

    def body(s_ref, idx_ref, scr):
        lin3 = (
            lax.broadcasted_iota(jnp.int32, (B, R, 128), 1) * 128
            + lax.broadcasted_iota(jnp.int32, (B, R, 128), 2)
        )
        scr[...] = jnp.where(lin3 < N, s_ref[...], NEG)
        lane64 = lax.broadcasted_iota(jnp.int32, (B, 64), 1)

        def step(j, idxacc):
            cur = scr[...]
            m4 = jnp.max(cur, axis=(1, 2))
            cand = jnp.where(
                cur == m4[:, None, None], lin3, jnp.int32(2147483647))
            amin4 = jnp.min(cand, axis=(1, 2))
            scr[...] = jnp.where(lin3 == amin4[:, None, None], NEG, cur)
            return jnp.where(lane64 == j, amin4[:, None], idxacc)

        idx_ref[...] = lax.fori_loop(
            0, N_CTX, step, jnp.zeros((B, 64), jnp.int32))

    return pl.pallas_call(
        body,
        in_specs=[pl.BlockSpec((B, R, 128), lambda: (0, 0, 0))],
        out_shape=jax.ShapeDtypeStruct((B, 64), jnp.int32),
        scratch_shapes=[pltpu.VMEM((B, R, 128), jnp.float32)],
    )(scores3)


def _logits_sc(idx50, clen_pad, cwf, a0, u, emb_rc, emb_feat):
    @functools.partial(
        pl.kernel,
        out_type=jax.ShapeDtypeStruct((B, PARTS, 16), jnp.float32),
        mesh=_MESH,
        scratch_types=[
            pltpu.VMEM((64,), jnp.int32),
            pltpu.VMEM((N_PAD,), jnp.int32),
            pltpu.VMEM((SENT_PER_PART, 128), jnp.int32),
            pltpu.VMEM((TOK,), jnp.int32),
            pltpu.VMEM((TOK,), jnp.int32),
            pltpu.VMEM((TOK, D), jnp.float32),
            pltpu.VMEM((TOK, D), jnp.float32),
            pltpu.VMEM((D,), jnp.float32),
            pltpu.VMEM((TOK,), jnp.float32),
            pltpu.VMEM((80,), jnp.int32),
            pltpu.VMEM((80,), jnp.int32),
            pltpu.VMEM((16,), jnp.int32),
            pltpu.VMEM((16,), jnp.float32),
            pltpu.SemaphoreType.DMA,
        ],
        compiler_params=_SC_PARAMS,
    )
    def k(idx_hbm, clen_hbm, cwf_hbm, a_hbm, u_hbm, rc_hbm, ft_hbm,
          out_hbm, idx_v, clen_v, id_rows, wflat, fflat, erc, eft,
          ubuf, zbuf, clbuf, stbuf, a_v, outv, sem):
        wid = _wid()
        b = wid // PARTS
        part = wid % PARTS
        pltpu.sync_copy(idx_hbm.at[b], idx_v)
        pltpu.sync_copy(clen_hbm, clen_v)
        pltpu.sync_copy(u_hbm.at[b], ubuf)
        pltpu.sync_copy(a_hbm, a_v)

        i16 = lax.broadcasted_iota(jnp.int32, (16,), 0)
        sent_ids = idx_v.at[pl.ds(part * SENT_PER_PART, SENT_PER_PART)]
        pltpu.async_copy(cwf_hbm.at[sent_ids], id_rows, sem).wait()
        for r in range(SENT_PER_PART):
            for h in range(2):
                wflat[pl.ds(r * LC + h * 16, 16)] = id_rows[
                    r, pl.ds(h * 16, 16)]
                fflat[pl.ds(r * LC + h * 16, 16)] = id_rows[
                    r, pl.ds(LC + h * 16, 16)]
        pltpu.async_copy(rc_hbm.at[wflat], erc, sem).wait()
        pltpu.async_copy(ft_hbm.at[fflat], eft, sem).wait()

        carry = jnp.int32(0)
        for kk in range(4):
            idv = idx_v[pl.ds(kk * 16, 16)]
            cl = plsc.load_gather(clen_v, [idv])
            cl = jnp.where(i16 + kk * 16 < N_CTX, cl, 0)
            inc = plsc.cumsum(cl)
            clbuf[pl.ds(kk * 16, 16)] = cl
            stbuf[pl.ds(kk * 16, 16)] = inc - cl + carry
            carry = carry + jnp.sum(cl)

        uv = [ubuf[pl.ds(h * 16, 16)] for h in range(8)]

        @pl.loop(0, TOK // 16)
        def _tok(g):
            zvec = jnp.zeros((16,), jnp.float32)
            for tloc in range(16):
                t = g * 16 + tloc
                acc = jnp.zeros((16,), jnp.float32)
                for h in range(8):
                    sl = pl.ds(h * 16, 16)
                    acc = acc + (erc[t, sl] + eft[t, sl]) * uv[h]
                zvec = zvec + jnp.where(i16 == tloc, jnp.sum(acc), 0.0)
            zbuf[pl.ds(g * 16, 16)] = zvec

        mybase = part * SENT_PER_PART
        clv = clbuf[pl.ds(mybase, 16)]
        stv = stbuf[pl.ds(mybase, 16)]

        runmax = jnp.full((16,), NEG, jnp.float32)
        for jj in range(SENT_PER_PART):
            lenj = clv[jj]
            for h in range(2):
                zv = zbuf[pl.ds(jj * LC + h * 16, 16)]
                valid = i16 + h * 16 < lenj
                runmax = jnp.maximum(runmax, jnp.where(valid, zv, NEG))
        m_loc = jnp.max(runmax)
        sacc = jnp.zeros((16,), jnp.float32)
        for jj in range(SENT_PER_PART):
            lenj = clv[jj]
            for h in range(2):
                zv = zbuf[pl.ds(jj * LC + h * 16, 16)]
                valid = i16 + h * 16 < lenj
                sacc = sacc + jnp.where(valid, jnp.exp(zv - m_loc), 0.0)
        s_loc = jnp.sum(sacc)

        av = a_v[...]
        p = jnp.sum(jnp.where(i16 == b, av, 0))
        zpart = jnp.float32(0.0)
        for jj in range(SENT_PER_PART):
            stj = stv[jj]
            lenj = clv[jj]
            own = (p >= stj) & (p < stj + lenj)
            off = p - stj
            zv0 = zbuf[pl.ds(jj * LC, 16)]
            zv1 = zbuf[pl.ds(jj * LC + 16, 16)]
            zt = (jnp.sum(jnp.where(i16 == off, zv0, 0.0))
                  + jnp.sum(jnp.where(i16 == off - 16, zv1, 0.0)))
            zpart = zpart + jnp.where(own, zt, 0.0)

        ov = jnp.where(
            i16 == 0, m_loc,
            jnp.where(i16 == 1, s_loc, jnp.where(i16 == 2, zpart, 0.0)),
        )
        outv[...] = ov
        pltpu.sync_copy(outv, out_hbm.at[b, part])

    return k(idx50, clen_pad, cwf, a0, u, emb_rc, emb_feat)


def _final_tc(e):
    def body(e_ref, o_ref):
        ev = e_ref[...]
        m = ev[:, :, 0]
        s = ev[:, :, 1]
        z = ev[:, :, 2]
        M = jnp.max(m, axis=1, keepdims=True)
        sm = (s * jnp.exp(m - M)).sum(1, keepdims=True)
        lse = jnp.log(sm) + M
        zt = z.sum(1, keepdims=True)
        o_ref[0, 0] = jnp.mean(lse - zt)

    return pl.pallas_call(
        body,
        in_specs=[pl.BlockSpec((B, PARTS, 16), lambda: (0, 0, 0))],
        out_specs=pl.BlockSpec(memory_space=pltpu.SMEM),
        out_shape=jax.ShapeDtypeStruct((1, 1), jnp.float32),
    )(e)


def kernel(q, c, a, qlen, clen, alen, c_batch_size, emb_ir, emb_rc, emb_feat,
           W_s, W_e):
    q = q.astype(jnp.int32)
    c = c.astype(jnp.int32)
    qlen = jnp.maximum(qlen.astype(jnp.int32), 1)
    clen = jnp.maximum(clen.astype(jnp.int32), 1)

    qw = q[:, :, 0]
    qf = q[:, :, 1]
    qe = _qgather_sc(qw, qf, emb_ir, emb_rc, emb_feat)

    qm = (jnp.arange(LQ)[None, :] < qlen[:, None]).astype(jnp.float32)
    inv_q = 1.0 / qm.sum(1, keepdims=True)
    T, u = _prep_tc(qe, qm, inv_q, W_s, emb_ir)

    c0 = c[:, :, 0]
    c0p = jnp.pad(c0, ((0, N_PAD - N), (0, 0)))
    clen_pad = jnp.pad(clen, (0, N_PAD - N), constant_values=1)
    c0m = jnp.where(jnp.arange(LC)[None, :] < clen_pad[:, None], c0p, V)
    c0t = c0m.reshape(N_PAD // 16, 16, LC).transpose(0, 2, 1).reshape(
        N_PAD // 16, LC * 16)

    inv_pad = 1.0 / clen_pad.astype(jnp.float32)
    scores = _score_sc(T, c0t, inv_pad)
    idx50 = _topk_tc(scores.reshape(B, N_PAD // 128, 128))

    a0 = jnp.pad(a[:, 0].astype(jnp.int32), (0, 16 - B))
    cwf = jnp.pad(jnp.concatenate([c0, c[:, :, 1]], axis=1), ((0, 0), (0, 64)))
    e = _logits_sc(idx50, clen_pad, cwf, a0, u, emb_rc, emb_feat)
    return _final_tc(e).reshape(())


# speedup vs baseline: 3.3203x; 1.0053x over previous
import functools

import jax
import jax.numpy as jnp
from jax import lax
from jax.experimental import pallas as pl
from jax.experimental.pallas import tpu as pltpu
from jax.experimental.pallas import tpu_sc as plsc

V = 50000
D = 128
N_CTX = 50
B, LQ, N, LC = 4, 32, 20000, 32

N_PAD = 20480
PARTS = 8
SENT_PER_W = N_PAD // PARTS
GROUPS_PER_W = SENT_PER_W // 16
CHUNK_GROUPS = 32
N_CHUNKS = GROUPS_PER_W // CHUNK_GROUPS
VB = 6272
V_T = 50176
SENT_PER_PART = 8
TOK = SENT_PER_PART * LC

_MESH = plsc.VectorSubcoreMesh(core_axis_name="c", subcore_axis_name="s")
_SC_PARAMS = pltpu.CompilerParams(needs_layout_passes=False)
NEG = -3e38


def _wid():
    return lax.axis_index("s") * 2 + lax.axis_index("c")


def _qgather_sc(qw, qf, emb_ir, emb_rc, emb_feat):
    @functools.partial(
        pl.kernel,
        out_type=jax.ShapeDtypeStruct((3, B * LQ, D), jnp.float32),
        mesh=_MESH,
        scratch_types=[
            pltpu.VMEM((LQ,), jnp.int32),
            pltpu.VMEM((LQ, D), jnp.float32),
            pltpu.SemaphoreType.DMA,
        ],
    )
    def k(qw_hbm, qf_hbm, ir_hbm, rc_hbm, ft_hbm, out_hbm, idx_v, rows_v, sem):
        wid = _wid()
        for tt, (ids_hbm, tab) in enumerate(
            [(qw_hbm, ir_hbm), (qw_hbm, rc_hbm), (qf_hbm, ft_hbm)]
        ):
            @pl.when((wid >= tt * B) & (wid < tt * B + B))
            def _(tt=tt, ids_hbm=ids_hbm, tab=tab):
                b = wid - tt * B
                pltpu.sync_copy(ids_hbm.at[b], idx_v)
                pltpu.async_copy(tab.at[idx_v], rows_v, sem).wait()
                pltpu.sync_copy(rows_v, out_hbm.at[tt, pl.ds(b * LQ, LQ)])

    return k(qw, qf, emb_ir, emb_rc, emb_feat)


def _prep_tc(qe, qm, inv_q, W_s, emb_ir):
    def body(qe_ref, qm_ref, inv_ref, ws_ref, emb_ref, T_ref, u_ref, qp_scr):
        i = pl.program_id(0)

        @pl.when(i == 0)
        def _():
            qe_all = qe_ref[...]
            qm3 = qm_ref[...][:, :, None]
            inv = inv_ref[...]
            qe_ir = qe_all[0].reshape(B, LQ, D)
            qe_rc = (qe_all[1] + qe_all[2]).reshape(B, LQ, D)
            qvec = (qe_ir * qm3).sum(1) * inv
            qv2 = (qe_rc * qm3).sum(1) * inv
            zpad = jnp.zeros((8 - B, D), jnp.float32)
            qp_scr[...] = jnp.concatenate([qvec, zpad], 0)
            uu = lax.dot_general(qv2, ws_ref[...], (((1,), (1,)), ((), ())))
            u_ref[...] = jnp.concatenate([uu, zpad], 0)

        cols = lax.broadcasted_iota(jnp.int32, (8, VB), 1) + i * VB
        vals = lax.dot_general(
            qp_scr[...], emb_ref[...], (((1,), (1,)), ((), ()))
        )
        T_ref[...] = jnp.where(cols < V, vals, 0.0)

    return pl.pallas_call(
        body,
        grid=(V_T // VB,),
        in_specs=[
            pl.BlockSpec((3, B * LQ, D), lambda i: (0, 0, 0)),
            pl.BlockSpec((B, LQ), lambda i: (0, 0)),
            pl.BlockSpec((B, 1), lambda i: (0, 0)),
            pl.BlockSpec((D, D), lambda i: (0, 0)),
            pl.BlockSpec((VB, D), lambda i: (i, 0)),
        ],
        out_specs=[
            pl.BlockSpec((8, VB), lambda i: (0, i)),
            pl.BlockSpec((8, D), lambda i: (0, 0)),
        ],
        out_shape=[
            jax.ShapeDtypeStruct((8, V_T), jnp.float32),
            jax.ShapeDtypeStruct((8, D), jnp.float32),
        ],
        scratch_shapes=[pltpu.VMEM((8, D), jnp.float32)],
    )(qe, qm, inv_q, W_s, emb_ir)


def _score_sc(T, c0t, inv_pad):
    CG16 = CHUNK_GROUPS * 16
    CROWS = CG16 // 4

    @functools.partial(
        pl.kernel,
        out_type=jax.ShapeDtypeStruct((B, N_PAD), jnp.float32),
        mesh=_MESH,
        scratch_types=[
            pltpu.VMEM((V_T,), jnp.float32),
            pltpu.VMEM((2, CHUNK_GROUPS, LC * 16), jnp.int32),
            pltpu.VMEM((2, CG16), jnp.float32),
            pltpu.VMEM((CG16,), jnp.float32),
            pltpu.SemaphoreType.DMA,
            pltpu.SemaphoreType.DMA,
        ],
        compiler_params=_SC_PARAMS,
    )
    def k(T_hbm, c0t_hbm, inv_hbm, out_hbm, tb_v, c_v, inv_v, o_v,
          sem0, sem1):
        wid = _wid()
        b = wid // PARTS
        part = wid % PARTS
        pltpu.sync_copy(T_hbm.at[b], tb_v)
        base_g = part * GROUPS_PER_W
        sems = [sem0, sem1]

        def start(ci):
            g0 = base_g + ci * CHUNK_GROUPS
            buf = ci % 2
            return (
                pltpu.async_copy(
                    c0t_hbm.at[pl.ds(g0, CHUNK_GROUPS)], c_v.at[buf],
                    sems[buf]),
                pltpu.async_copy(
                    inv_hbm.at[pl.ds(g0 * 16, CG16)], inv_v.at[buf],
                    sems[buf]),
            )

        pend = start(0)
        for ci in range(N_CHUNKS):
            buf = ci % 2
            pend[0].wait()
            pend[1].wait()
            if ci + 1 < N_CHUNKS:
                pend = start(ci + 1)

            @pl.loop(0, CHUNK_GROUPS)
            def _grp(gl, buf=buf):
                acc0 = jnp.zeros((16,), jnp.float32)
                acc1 = jnp.zeros((16,), jnp.float32)
                for l in range(0, LC, 2):
                    acc0 = acc0 + plsc.load_gather(
                        tb_v, [c_v[buf, gl, pl.ds(l * 16, 16)]])
                    acc1 = acc1 + plsc.load_gather(
                        tb_v, [c_v[buf, gl, pl.ds((l + 1) * 16, 16)]])
                o_v[pl.ds(gl * 16, 16)] = (acc0 + acc1) * inv_v[
                    buf, pl.ds(gl * 16, 16)]

            col0 = part * SENT_PER_W + ci * CG16
            pltpu.sync_copy(o_v, out_hbm.at[b, pl.ds(col0, CG16)])

    return k(T, c0t, inv_pad)


def _topk_tc(scores3):
    R = N_PAD // 128

    def body(s_ref, idx_ref, scr):
        lin3 = (
            lax.broadcasted_iota(jnp.int32, (B, R, 128), 1) * 128
            + lax.broadcasted_iota(jnp.int32, (B, R, 128), 2)
        )
        scr[...] = jnp.where(lin3 < N, s_ref[...], NEG)
        lane64 = lax.broadcasted_iota(jnp.int32, (B, 64), 1)

        def step(j, idxacc):
            cur = scr[...]
            m4 = jnp.max(cur, axis=(1, 2))
            cand = jnp.where(
                cur == m4[:, None, None], lin3, jnp.int32(2147483647))
            amin4 = jnp.min(cand, axis=(1, 2))
            scr[...] = jnp.where(lin3 == amin4[:, None, None], NEG, cur)
            return jnp.where(lane64 == j, amin4[:, None], idxacc)

        idx_ref[...] = lax.fori_loop(
            0, N_CTX, step, jnp.zeros((B, 64), jnp.int32))

    return pl.pallas_call(
        body,
        in_specs=[pl.BlockSpec((B, R, 128), lambda: (0, 0, 0))],
        out_shape=jax.ShapeDtypeStruct((B, 64), jnp.int32),
        scratch_shapes=[pltpu.VMEM((B, R, 128), jnp.float32)],
    )(scores3)


def _logits_sc(idx50, clen_pad, cwf, a0, u, emb_rc, emb_feat):
    @functools.partial(
        pl.kernel,
        out_type=jax.ShapeDtypeStruct((B, PARTS, 16), jnp.float32),
        mesh=_MESH,
        scratch_types=[
            pltpu.VMEM((64,), jnp.int32),
            pltpu.VMEM((N_PAD,), jnp.int32),
            pltpu.VMEM((SENT_PER_PART, 128), jnp.int32),
            pltpu.VMEM((TOK,), jnp.int32),
            pltpu.VMEM((TOK,), jnp.int32),
            pltpu.VMEM((TOK, D), jnp.float32),
            pltpu.VMEM((TOK, D), jnp.float32),
            pltpu.VMEM((D,), jnp.float32),
            pltpu.VMEM((TOK,), jnp.float32),
            pltpu.VMEM((80,), jnp.int32),
            pltpu.VMEM((80,), jnp.int32),
            pltpu.VMEM((16,), jnp.int32),
            pltpu.VMEM((16,), jnp.float32),
            pltpu.SemaphoreType.DMA,
        ],
        compiler_params=_SC_PARAMS,
    )
    def k(idx_hbm, clen_hbm, cwf_hbm, a_hbm, u_hbm, rc_hbm, ft_hbm,
          out_hbm, idx_v, clen_v, id_rows, wflat, fflat, erc, eft,
          ubuf, zbuf, clbuf, stbuf, a_v, outv, sem):
        wid = _wid()
        b = wid // PARTS
        part = wid % PARTS
        pltpu.sync_copy(idx_hbm.at[b], idx_v)
        pltpu.sync_copy(clen_hbm, clen_v)
        pltpu.sync_copy(u_hbm.at[b], ubuf)
        pltpu.sync_copy(a_hbm, a_v)

        i16 = lax.broadcasted_iota(jnp.int32, (16,), 0)
        sent_ids = idx_v.at[pl.ds(part * SENT_PER_PART, SENT_PER_PART)]
        pltpu.async_copy(cwf_hbm.at[sent_ids], id_rows, sem).wait()
        for r in range(SENT_PER_PART):
            for h in range(2):
                wflat[pl.ds(r * LC + h * 16, 16)] = id_rows[
                    r, pl.ds(h * 16, 16)]
                fflat[pl.ds(r * LC + h * 16, 16)] = id_rows[
                    r, pl.ds(LC + h * 16, 16)]
        pltpu.async_copy(rc_hbm.at[wflat], erc, sem).wait()
        pltpu.async_copy(ft_hbm.at[fflat], eft, sem).wait()

        carry = jnp.int32(0)
        for kk in range(4):
            idv = idx_v[pl.ds(kk * 16, 16)]
            cl = plsc.load_gather(clen_v, [idv])
            cl = jnp.where(i16 + kk * 16 < N_CTX, cl, 0)
            inc = plsc.cumsum(cl)
            clbuf[pl.ds(kk * 16, 16)] = cl
            stbuf[pl.ds(kk * 16, 16)] = inc - cl + carry
            carry = carry + jnp.sum(cl)

        uv = [ubuf[pl.ds(h * 16, 16)] for h in range(8)]

        @pl.loop(0, TOK // 16)
        def _tok(g):
            zvec = jnp.zeros((16,), jnp.float32)
            for tloc in range(16):
                t = g * 16 + tloc
                acc = jnp.zeros((16,), jnp.float32)
                for h in range(8):
                    sl = pl.ds(h * 16, 16)
                    acc = acc + (erc[t, sl] + eft[t, sl]) * uv[h]
                zvec = zvec + jnp.where(i16 == tloc, jnp.sum(acc), 0.0)
            zbuf[pl.ds(g * 16, 16)] = zvec

        mybase = part * SENT_PER_PART
        clv = clbuf[pl.ds(mybase, 16)]
        stv = stbuf[pl.ds(mybase, 16)]

        runmax = jnp.full((16,), NEG, jnp.float32)
        for jj in range(SENT_PER_PART):
            lenj = clv[jj]
            for h in range(2):
                zv = zbuf[pl.ds(jj * LC + h * 16, 16)]
                valid = i16 + h * 16 < lenj
                runmax = jnp.maximum(runmax, jnp.where(valid, zv, NEG))
        m_loc = jnp.max(runmax)
        sacc = jnp.zeros((16,), jnp.float32)
        for jj in range(SENT_PER_PART):
            lenj = clv[jj]
            for h in range(2):
                zv = zbuf[pl.ds(jj * LC + h * 16, 16)]
                valid = i16 + h * 16 < lenj
                sacc = sacc + jnp.where(valid, jnp.exp(zv - m_loc), 0.0)
        s_loc = jnp.sum(sacc)

        av = a_v[...]
        p = jnp.sum(jnp.where(i16 == b, av, 0))
        zpart = jnp.float32(0.0)
        for jj in range(SENT_PER_PART):
            stj = stv[jj]
            lenj = clv[jj]
            own = (p >= stj) & (p < stj + lenj)
            off = p - stj
            zv0 = zbuf[pl.ds(jj * LC, 16)]
            zv1 = zbuf[pl.ds(jj * LC + 16, 16)]
            zt = (jnp.sum(jnp.where(i16 == off, zv0, 0.0))
                  + jnp.sum(jnp.where(i16 == off - 16, zv1, 0.0)))
            zpart = zpart + jnp.where(own, zt, 0.0)

        ov = jnp.where(
            i16 == 0, m_loc,
            jnp.where(i16 == 1, s_loc, jnp.where(i16 == 2, zpart, 0.0)),
        )
        outv[...] = ov
        pltpu.sync_copy(outv, out_hbm.at[b, part])

    return k(idx50, clen_pad, cwf, a0, u, emb_rc, emb_feat)


def _final_tc(e):
    def body(e_ref, o_ref):
        ev = e_ref[...]
        m = ev[:, :, 0]
        s = ev[:, :, 1]
        z = ev[:, :, 2]
        M = jnp.max(m, axis=1, keepdims=True)
        sm = (s * jnp.exp(m - M)).sum(1, keepdims=True)
        lse = jnp.log(sm) + M
        zt = z.sum(1, keepdims=True)
        o_ref[0, 0] = jnp.mean(lse - zt)

    return pl.pallas_call(
        body,
        in_specs=[pl.BlockSpec((B, PARTS, 16), lambda: (0, 0, 0))],
        out_specs=pl.BlockSpec(memory_space=pltpu.SMEM),
        out_shape=jax.ShapeDtypeStruct((1, 1), jnp.float32),
    )(e)


def kernel(q, c, a, qlen, clen, alen, c_batch_size, emb_ir, emb_rc, emb_feat,
           W_s, W_e):
    q = q.astype(jnp.int32)
    c = c.astype(jnp.int32)
    qlen = jnp.maximum(qlen.astype(jnp.int32), 1)
    clen = jnp.maximum(clen.astype(jnp.int32), 1)

    qw = q[:, :, 0]
    qf = q[:, :, 1]
    qe = _qgather_sc(qw, qf, emb_ir, emb_rc, emb_feat)

    qm = (jnp.arange(LQ)[None, :] < qlen[:, None]).astype(jnp.float32)
    inv_q = 1.0 / qm.sum(1, keepdims=True)
    T, u = _prep_tc(qe, qm, inv_q, W_s, emb_ir)

    c0 = c[:, :, 0]
    c0p = jnp.pad(c0, ((0, N_PAD - N), (0, 0)))
    clen_pad = jnp.pad(clen, (0, N_PAD - N), constant_values=1)
    c0m = jnp.where(jnp.arange(LC)[None, :] < clen_pad[:, None], c0p, V)
    c0t = c0m.reshape(N_PAD // 16, 16, LC).transpose(0, 2, 1).reshape(
        N_PAD // 16, LC * 16)

    inv_pad = 1.0 / clen_pad.astype(jnp.float32)
    scores = _score_sc(T, c0t, inv_pad)
    idx50 = _topk_tc(scores.reshape(B, N_PAD // 128, 128))

    a0 = jnp.pad(a[:, 0].astype(jnp.int32), (0, 16 - B))
    cwf = jnp.pad(jnp.concatenate([c0, c[:, :, 1]], axis=1), ((0, 0), (0, 64)))
    e = _logits_sc(idx50, clen_pad, cwf, a0, u, emb_rc, emb_feat)
    return _final_tc(e).reshape(())
